# initial kernel scaffold (unmeasured)
import jax
import jax.numpy as jnp
from jax import lax
from jax.experimental import pallas as pl
from jax.experimental.pallas import tpu as pltpu

N_DEV = 16
M_BLK = 256
BN = 1024


def kernel(x, w_mat, scale_x, scale_w):
    m, k_per = x.shape
    k, n = w_mat.shape
    assert m == N_DEV * M_BLK and k_per == M_BLK
    scale = (scale_x * scale_w).reshape(1, 1).astype(jnp.float32)
    n_steps = n // BN

    def body(x_ref, w_ref, scale_ref, out_ref, gather_ref, send_sems, recv_sems):
        my = lax.axis_index("i")

        @pl.when(pl.program_id(0) == 0)
        def _():
            barrier = pltpu.get_barrier_semaphore()
            for o in range(1, N_DEV):
                pl.semaphore_signal(
                    barrier, inc=1,
                    device_id=((my + o) % N_DEV,),
                    device_id_type=pl.DeviceIdType.MESH,
                )
            pl.semaphore_wait(barrier, N_DEV - 1)

            gather_ref[:, pl.ds(my * M_BLK, M_BLK)] = x_ref[pl.ds(my * M_BLK, M_BLK), :]

            sends = []
            for idx, o in enumerate(range(1, N_DEV)):
                p = (my + o) % N_DEV
                rdma = pltpu.make_async_remote_copy(
                    src_ref=x_ref.at[pl.ds(p * M_BLK, M_BLK), :],
                    dst_ref=gather_ref.at[:, pl.ds(my * M_BLK, M_BLK)],
                    send_sem=send_sems.at[idx],
                    recv_sem=recv_sems.at[idx],
                    device_id=(p,),
                    device_id_type=pl.DeviceIdType.MESH,
                )
                rdma.start()
                sends.append(rdma)

            for idx, o in enumerate(range(1, N_DEV)):
                s = (my - o) % N_DEV
                recv = pltpu.make_async_remote_copy(
                    src_ref=x_ref.at[pl.ds(0, M_BLK), :],
                    dst_ref=gather_ref.at[:, pl.ds(s * M_BLK, M_BLK)],
                    send_sem=send_sems.at[idx],
                    recv_sem=recv_sems.at[idx],
                    device_id=(s,),
                    device_id_type=pl.DeviceIdType.MESH,
                )
                recv.wait_recv()
            for rdma in sends:
                rdma.wait_send()

        acc = jnp.dot(gather_ref[:, :], w_ref[:, :],
                      preferred_element_type=jnp.float32)
        y = acc * scale_ref[0, 0]
        out_ref[:, :] = y * jax.nn.sigmoid(y)

    return pl.pallas_call(
        body,
        grid=(n_steps,),
        in_specs=[
            pl.BlockSpec((m, k_per), lambda j: (0, 0)),
            pl.BlockSpec((k, BN), lambda j: (0, j)),
            pl.BlockSpec(memory_space=pltpu.SMEM),
        ],
        out_specs=pl.BlockSpec((M_BLK, BN), lambda j: (0, j)),
        out_shape=jax.ShapeDtypeStruct((M_BLK, n), jnp.float32),
        scratch_shapes=[
            pltpu.VMEM((M_BLK, k), x.dtype),
            pltpu.SemaphoreType.DMA((N_DEV - 1,)),
            pltpu.SemaphoreType.DMA((N_DEV - 1,)),
        ],
        compiler_params=pltpu.CompilerParams(collective_id=0),
    )(x, w_mat, scale)


# baseline (device time: 67858 ns/iter reference)
import jax
import jax.numpy as jnp
from jax import lax
from jax.experimental import pallas as pl
from jax.experimental.pallas import tpu as pltpu

N_DEV = 16
M_BLK = 256
BN = 512
F8 = jnp.float8_e4m3fn


def kernel(x, w_mat, scale_x, scale_w):
    m, k_per = x.shape
    k, n = w_mat.shape
    assert m == N_DEV * M_BLK and k_per == M_BLK
    scale = (scale_x * scale_w).reshape(1, 1).astype(jnp.float32)
    n_steps = n // BN

    def body(x_ref, w_ref, scale_ref, out_ref,
             x8_ref, gather_ref, send_sems, recv_sems):
        my = lax.axis_index("i")

        @pl.when(pl.program_id(0) == 0)
        def _():
            x8_ref[:, :] = x_ref[:, :].astype(F8)

            barrier = pltpu.get_barrier_semaphore()
            for o in range(1, N_DEV):
                pl.semaphore_signal(
                    barrier, inc=1,
                    device_id=((my + o) % N_DEV,),
                    device_id_type=pl.DeviceIdType.MESH,
                )
            pl.semaphore_wait(barrier, N_DEV - 1)

            gather_ref[:, pl.ds(my * M_BLK, M_BLK)] = x8_ref[pl.ds(my * M_BLK, M_BLK), :]

            sends = []
            for idx, o in enumerate(range(1, N_DEV)):
                p = (my + o) % N_DEV
                rdma = pltpu.make_async_remote_copy(
                    src_ref=x8_ref.at[pl.ds(p * M_BLK, M_BLK), :],
                    dst_ref=gather_ref.at[:, pl.ds(my * M_BLK, M_BLK)],
                    send_sem=send_sems.at[idx],
                    recv_sem=recv_sems.at[idx],
                    device_id=(p,),
                    device_id_type=pl.DeviceIdType.MESH,
                )
                rdma.start()
                sends.append(rdma)

            for idx, o in enumerate(range(1, N_DEV)):
                s = (my - o) % N_DEV
                recv = pltpu.make_async_remote_copy(
                    src_ref=x8_ref.at[pl.ds(0, M_BLK), :],
                    dst_ref=gather_ref.at[:, pl.ds(s * M_BLK, M_BLK)],
                    send_sem=send_sems.at[idx],
                    recv_sem=recv_sems.at[idx],
                    device_id=(s,),
                    device_id_type=pl.DeviceIdType.MESH,
                )
                recv.wait_recv()
            for rdma in sends:
                rdma.wait_send()

        acc = jnp.dot(gather_ref[:, :], w_ref[:, :].astype(F8),
                      preferred_element_type=jnp.float32)
        y = acc * scale_ref[0, 0]
        out_ref[:, :] = y * jax.nn.sigmoid(y)

    return pl.pallas_call(
        body,
        grid=(n_steps,),
        in_specs=[
            pl.BlockSpec((m, k_per), lambda j: (0, 0)),
            pl.BlockSpec((k, BN), lambda j: (0, j)),
            pl.BlockSpec(memory_space=pltpu.SMEM),
        ],
        out_specs=pl.BlockSpec((M_BLK, BN), lambda j: (0, j)),
        out_shape=jax.ShapeDtypeStruct((M_BLK, n), jnp.float32),
        scratch_shapes=[
            pltpu.VMEM((m, k_per), F8),
            pltpu.VMEM((M_BLK, k), F8),
            pltpu.SemaphoreType.DMA((N_DEV - 1,)),
            pltpu.SemaphoreType.DMA((N_DEV - 1,)),
        ],
        compiler_params=pltpu.CompilerParams(
            collective_id=0,
            vmem_limit_bytes=64 * 1024 * 1024,
        ),
    )(x, w_mat, scale)


# device time: 67030 ns/iter; 1.0124x vs baseline; 1.0124x over previous
import jax
import jax.numpy as jnp
from jax import lax
from jax.experimental import pallas as pl
from jax.experimental.pallas import tpu as pltpu

N_DEV = 16
M_BLK = 256
F8 = jnp.float8_e4m3fn


def kernel(x, w_mat, scale_x, scale_w):
    m, k_per = x.shape
    k, n = w_mat.shape
    assert m == N_DEV * M_BLK and k_per == M_BLK and k == N_DEV * M_BLK
    scale = (scale_x * scale_w).reshape(1, 1).astype(jnp.float32)

    def body(x_ref, w_ref, scale_ref, out_ref,
             x8_ref, gather_ref, send_sems, recv_sems):
        my = lax.axis_index("i")
        t = pl.program_id(0)

        @pl.when(t == 0)
        def _():
            x8_ref[:, :] = x_ref[:, :].astype(F8)

            barrier = pltpu.get_barrier_semaphore()
            for o in range(1, N_DEV):
                pl.semaphore_signal(
                    barrier, inc=1,
                    device_id=((my + o) % N_DEV,),
                    device_id_type=pl.DeviceIdType.MESH,
                )
            pl.semaphore_wait(barrier, N_DEV - 1)

            gather_ref[:, pl.ds(my * M_BLK, M_BLK)] = x8_ref[pl.ds(my * M_BLK, M_BLK), :]

            for o in range(1, N_DEV):
                p = (my + o) % N_DEV
                rdma = pltpu.make_async_remote_copy(
                    src_ref=x8_ref.at[pl.ds(p * M_BLK, M_BLK), :],
                    dst_ref=gather_ref.at[:, pl.ds(my * M_BLK, M_BLK)],
                    send_sem=send_sems.at[p],
                    recv_sem=recv_sems.at[my],
                    device_id=(p,),
                    device_id_type=pl.DeviceIdType.MESH,
                )
                rdma.start()

        @pl.when(t != my)
        def _():
            recv = pltpu.make_async_remote_copy(
                src_ref=x8_ref.at[pl.ds(0, M_BLK), :],
                dst_ref=gather_ref.at[:, pl.ds(t * M_BLK, M_BLK)],
                send_sem=send_sems.at[t],
                recv_sem=recv_sems.at[t],
                device_id=(t,),
                device_id_type=pl.DeviceIdType.MESH,
            )
            recv.wait_recv()

        partial = jnp.dot(
            gather_ref[:, pl.ds(t * M_BLK, M_BLK)],
            w_ref[:, :].astype(F8),
            preferred_element_type=jnp.float32,
        )

        @pl.when(t == 0)
        def _():
            out_ref[:, :] = partial

        @pl.when(t != 0)
        def _():
            out_ref[:, :] += partial

        @pl.when(t == N_DEV - 1)
        def _():
            for o in range(1, N_DEV):
                p = (my + o) % N_DEV
                send = pltpu.make_async_remote_copy(
                    src_ref=x8_ref.at[pl.ds(p * M_BLK, M_BLK), :],
                    dst_ref=gather_ref.at[:, pl.ds(my * M_BLK, M_BLK)],
                    send_sem=send_sems.at[p],
                    recv_sem=recv_sems.at[my],
                    device_id=(p,),
                    device_id_type=pl.DeviceIdType.MESH,
                )
                send.wait_send()
            y = out_ref[:, :] * scale_ref[0, 0]
            out_ref[:, :] = y * jax.nn.sigmoid(y)

    return pl.pallas_call(
        body,
        grid=(N_DEV,),
        in_specs=[
            pl.BlockSpec((m, k_per), lambda t: (0, 0)),
            pl.BlockSpec((M_BLK, n), lambda t: (t, 0)),
            pl.BlockSpec(memory_space=pltpu.SMEM),
        ],
        out_specs=pl.BlockSpec((M_BLK, n), lambda t: (0, 0)),
        out_shape=jax.ShapeDtypeStruct((M_BLK, n), jnp.float32),
        scratch_shapes=[
            pltpu.VMEM((m, k_per), F8),
            pltpu.VMEM((M_BLK, k), F8),
            pltpu.SemaphoreType.DMA((N_DEV,)),
            pltpu.SemaphoreType.DMA((N_DEV,)),
        ],
        compiler_params=pltpu.CompilerParams(
            collective_id=0,
            vmem_limit_bytes=64 * 1024 * 1024,
        ),
    )(x, w_mat, scale)


# device time: 61951 ns/iter; 1.0953x vs baseline; 1.0820x over previous
import jax
import jax.numpy as jnp
from jax import lax
from jax.experimental import pallas as pl
from jax.experimental.pallas import tpu as pltpu

N_DEV = 16
M_BLK = 256
DEPTH = 3
F8 = jnp.float8_e4m3fn


def kernel(x, w_mat, scale_x, scale_w):
    m, k_per = x.shape
    k, n = w_mat.shape
    assert m == N_DEV * M_BLK and k_per == M_BLK and k == N_DEV * M_BLK
    scale = (scale_x * scale_w).reshape(1, 1).astype(jnp.float32)

    def body(x_ref, w_ref, scale_ref, out_ref,
             x8_ref, gather_ref, w_vmem, fetch_sems, send_sems, recv_sems):
        my = lax.axis_index("i")

        def w_fetch(d, slot):
            s = (my + d) % N_DEV
            return pltpu.make_async_copy(
                w_ref.at[pl.ds(s * M_BLK, M_BLK), :],
                w_vmem.at[slot],
                fetch_sems.at[slot],
            )

        for d in range(DEPTH):
            w_fetch(d, d % DEPTH).start()

        x8_ref[:, :] = x_ref[:, :].astype(F8)

        barrier = pltpu.get_barrier_semaphore()
        for o in range(1, N_DEV):
            pl.semaphore_signal(
                barrier, inc=1,
                device_id=((my + o) % N_DEV,),
                device_id_type=pl.DeviceIdType.MESH,
            )
        pl.semaphore_wait(barrier, N_DEV - 1)

        sends = []
        for o in range(1, N_DEV):
            p = (my + o) % N_DEV
            rdma = pltpu.make_async_remote_copy(
                src_ref=x8_ref.at[pl.ds(p * M_BLK, M_BLK), :],
                dst_ref=gather_ref.at[:, pl.ds(my * M_BLK, M_BLK)],
                send_sem=send_sems.at[p],
                recv_sem=recv_sems.at[my],
                device_id=(p,),
                device_id_type=pl.DeviceIdType.MESH,
            )
            rdma.start()
            sends.append(rdma)

        gather_ref[:, pl.ds(my * M_BLK, M_BLK)] = x8_ref[pl.ds(my * M_BLK, M_BLK), :]

        for d in range(N_DEV):
            s = (my + d) % N_DEV
            if d > 0:
                pltpu.make_async_remote_copy(
                    src_ref=x8_ref.at[pl.ds(0, M_BLK), :],
                    dst_ref=gather_ref.at[:, pl.ds(s * M_BLK, M_BLK)],
                    send_sem=send_sems.at[s],
                    recv_sem=recv_sems.at[s],
                    device_id=(s,),
                    device_id_type=pl.DeviceIdType.MESH,
                ).wait_recv()
            slot = d % DEPTH
            w_fetch(d, slot).wait()
            partial = jnp.dot(
                gather_ref[:, pl.ds(s * M_BLK, M_BLK)],
                w_vmem[slot].astype(F8),
                preferred_element_type=jnp.float32,
            )
            if d == 0:
                out_ref[:, :] = partial
            else:
                out_ref[:, :] += partial
            if d + DEPTH < N_DEV:
                w_fetch(d + DEPTH, slot).start()

        for rdma in sends:
            rdma.wait_send()
        y = out_ref[:, :] * scale_ref[0, 0]
        out_ref[:, :] = y * jax.nn.sigmoid(y)

    return pl.pallas_call(
        body,
        in_specs=[
            pl.BlockSpec(memory_space=pltpu.VMEM),
            pl.BlockSpec(memory_space=pl.ANY),
            pl.BlockSpec(memory_space=pltpu.SMEM),
        ],
        out_specs=pl.BlockSpec(memory_space=pltpu.VMEM),
        out_shape=jax.ShapeDtypeStruct((M_BLK, n), jnp.float32),
        scratch_shapes=[
            pltpu.VMEM((m, k_per), F8),
            pltpu.VMEM((M_BLK, k), F8),
            pltpu.VMEM((DEPTH, M_BLK, n), w_mat.dtype),
            pltpu.SemaphoreType.DMA((DEPTH,)),
            pltpu.SemaphoreType.DMA((N_DEV,)),
            pltpu.SemaphoreType.DMA((N_DEV,)),
        ],
        compiler_params=pltpu.CompilerParams(
            collective_id=0,
            vmem_limit_bytes=64 * 1024 * 1024,
        ),
    )(x, w_mat, scale)
